# baseline (speedup 1.0000x reference)
import jax
import jax.numpy as jnp
from jax.experimental import pallas as pl

N = 50000
E = 800000
T = 20
FEAT = 3
HID = 16
LAYERS = 3
GH = 16
GOUT = 64
IN_DIM = HID * LAYERS
PHID = 256
Q = 4096


def _gru(x, params):
    xs = jnp.transpose(x, (1, 0, 2))
    hs = []
    inp = xs
    for l in range(LAYERS):
        Wih = params['gru_Wih'][l]
        Whh = params['gru_Whh'][l]
        bih = params['gru_bih'][l]
        bhh = params['gru_bhh'][l]
        h0 = jnp.zeros((x.shape[0], HID), jnp.float32)

        def step(h, xt, Wih=Wih, Whh=Whh, bih=bih, bhh=bhh):
            gi = xt @ Wih.T + bih
            gh = h @ Whh.T + bhh
            i_r, i_z, i_n = jnp.split(gi, 3, axis=-1)
            h_r, h_z, h_n = jnp.split(gh, 3, axis=-1)
            r = jax.nn.sigmoid(i_r + h_r)
            z = jax.nn.sigmoid(i_z + h_z)
            n = jnp.tanh(i_n + r * h_n)
            hn = (1.0 - z) * n + z * h
            return hn, hn

        hT, ys = jax.lax.scan(step, h0, inp)
        inp = ys
        hs.append(hT)
    return jnp.stack(hs, axis=0)


def _gat_layer(feat, Wfc, Wa, src, dst, n):
    h = feat @ Wfc.T
    a_s = h @ Wa[0, : h.shape[1]]
    a_d = h @ Wa[0, h.shape[1]:]
    e = jax.nn.sigmoid(a_s[src] + a_d[dst])
    msg = e[:, None] * h[src]
    return jax.ops.segment_sum(msg, dst, num_segments=n)


def _mlp_kernel(union_ref, w1_ref, b1_ref, w2_ref, b2_ref, out_ref):
    z1 = jnp.maximum(union_ref[...] @ w1_ref[...] + b1_ref[...], 0.0)
    o = z1 @ w2_ref[...] + b2_ref[...]
    out_ref[...] = jax.nn.sigmoid(o)


def kernel(x, params, edge_index, q_from, q_to):
    src = edge_index[0]
    dst = edge_index[1]
    n = x.shape[0]
    hn = _gru(x, params)
    traj_feat = jnp.transpose(hn, (1, 0, 2)).reshape((n, -1))
    traj_feat2 = hn[-1]
    h1 = _gat_layer(traj_feat, params['fc'][0], params['att'][0], src, dst, n)
    h2 = _gat_layer(traj_feat, params['fc'][1], params['att'][1], src, dst, n)
    h3 = _gat_layer(traj_feat, params['fc'][2], params['att'][2], src, dst, n)
    h = jnp.concatenate([h1, h2, h3, h3], axis=1)
    h = jax.nn.elu(h)
    g_feat = _gat_layer(h, params['fc2'], params['att2'], src, dst, n)
    union = jnp.concatenate(
        [g_feat[q_from], g_feat[q_to], traj_feat2[q_from], traj_feat2[q_to]], axis=1)
    out = pl.pallas_call(
        _mlp_kernel,
        out_shape=jax.ShapeDtypeStruct((Q, 1), jnp.float32),
    )(union, params['pW1'].T, params['pb1'], params['pW2'].T, params['pb2'])
    return out


# speedup vs baseline: 5.3251x; 5.3251x over previous
import functools

import jax
import jax.numpy as jnp
from jax import lax
from jax.experimental import pallas as pl
from jax.experimental.pallas import tpu as pltpu
from jax.experimental.pallas import tpu_sc as plsc

HID = 16
GH = 16
GOUT = 64
CH = 128
NC = 2
NS = 16
NW = NC * NS
BN = 1024


def _f16_encode(x):
    b = lax.bitcast_convert_type(x, jnp.int32)
    sign = lax.shift_right_logical(b, 16) & jnp.int32(0x8000)
    mag = b & jnp.int32(0x7FFFFFFF)
    mag = jnp.clip(mag, jnp.int32(0x38800000), jnp.int32(0x477FE000))
    em = mag - jnp.int32(0x38000000)
    r = (em + jnp.int32(0x0FFF) + ((em >> 13) & jnp.int32(1))) >> 13
    return sign | r


def _pack_logits(a, d):
    return (_f16_encode(a) << 16) | _f16_encode(d)


def _f16_bits_to_f32(bits):
    sign = (bits & 0x8000) << 16
    em = bits & 0x7FFF
    fb = sign | ((em << 13) + 0x38000000)
    return plsc.bitcast(fb, jnp.float32)


def _prep_body(T, xtm_ref, wih0, wih1, wih2, whh0, whh1, whh2,
               bih0, bih1, bih2, bhh0, bhh1, bhh2,
               fcT0, fcT1, fcT2, asv0, asv1, asv2, adv0, adv1, adv2,
               h1t, h2t, h3t, p0, p1, p2, traj2):
    B = xtm_ref.shape[1]
    h = [jnp.zeros((B, HID), jnp.float32) for _ in range(3)]
    wih = (wih0, wih1, wih2)
    whh = (whh0, whh1, whh2)
    bih = (bih0, bih1, bih2)
    bhh = (bhh0, bhh1, bhh2)

    def gru_step(inp, hprev, l):
        gi = jnp.dot(inp, wih[l][...], preferred_element_type=jnp.float32) + bih[l][...]
        gh = jnp.dot(hprev, whh[l][...], preferred_element_type=jnp.float32) + bhh[l][...]
        r = jax.nn.sigmoid(gi[:, 0:HID] + gh[:, 0:HID])
        z = jax.nn.sigmoid(gi[:, HID:2 * HID] + gh[:, HID:2 * HID])
        n = jnp.tanh(gi[:, 2 * HID:] + r * gh[:, 2 * HID:])
        return (1.0 - z) * n + z * hprev

    for t in range(T):
        inp = xtm_ref[t]
        for l in range(3):
            h[l] = gru_step(inp, h[l], l)
            inp = h[l]

    tf = jnp.concatenate(h, axis=1)
    fcT = (fcT0, fcT1, fcT2)
    asv = (asv0, asv1, asv2)
    adv = (adv0, adv1, adv2)
    houts = (h1t, h2t, h3t)
    pouts = (p0, p1, p2)
    for l in range(3):
        H = jnp.dot(tf, fcT[l][...], preferred_element_type=jnp.float32)
        houts[l][...] = H
        a = jnp.dot(H, asv[l][...], preferred_element_type=jnp.float32)
        d = jnp.dot(H, adv[l][...], preferred_element_type=jnp.float32)
        pouts[l][...] = _pack_logits(a, d)
    traj2[...] = h[2]


def _run_prep(xtm, params, NP, T, FEAT):
    grid = (NP // BN,)
    full = lambda shape: pl.BlockSpec(shape, lambda i: (0,) * len(shape))
    row = lambda w: pl.BlockSpec((BN, w), lambda i: (i, 0))
    in_specs = [pl.BlockSpec((T, BN, FEAT), lambda i: (0, i, 0))]
    args = [xtm]
    for l in range(3):
        args.append(params['gru_Wih'][l].T)
        in_specs.append(full((FEAT if l == 0 else HID, 3 * HID)))
    for l in range(3):
        args.append(params['gru_Whh'][l].T)
        in_specs.append(full((HID, 3 * HID)))
    for l in range(3):
        args.append(params['gru_bih'][l].reshape(1, 3 * HID))
        in_specs.append(full((1, 3 * HID)))
    for l in range(3):
        args.append(params['gru_bhh'][l].reshape(1, 3 * HID))
        in_specs.append(full((1, 3 * HID)))
    for l in range(3):
        args.append(params['fc'][l].T)
        in_specs.append(full((3 * HID, GH)))
    for l in range(3):
        args.append(params['att'][l][0, :GH].reshape(GH, 1))
        in_specs.append(full((GH, 1)))
    for l in range(3):
        args.append(params['att'][l][0, GH:].reshape(GH, 1))
        in_specs.append(full((GH, 1)))
    out_shape = (
        [jax.ShapeDtypeStruct((NP, GH), jnp.float32)] * 3
        + [jax.ShapeDtypeStruct((NP, 1), jnp.int32)] * 3
        + [jax.ShapeDtypeStruct((NP, HID), jnp.float32)]
    )
    out_specs = [row(GH)] * 3 + [row(1)] * 3 + [row(HID)]
    return pl.pallas_call(
        functools.partial(_prep_body, T),
        grid=grid,
        in_specs=in_specs,
        out_specs=out_specs,
        out_shape=out_shape,
    )(*args)


def _edge_pass(tab_hbm, pk_hbm, zeros_hbm, src_hbm, dst_hbm,
               out_hbm, acc, pkv, sidx, didx, rows, msg, gsem,
               cid, sid, rows_per, nch, load_table):
    D = tab_hbm.shape[1]
    if load_table:
        pltpu.sync_copy(pk_hbm, pkv)
    row0 = sid * rows_per
    pltpu.sync_copy(zeros_hbm.at[pl.ds(row0, rows_per)],
                    acc.at[pl.ds(row0, rows_per)])
    plsc.subcore_barrier()

    wid = sid * NC + cid

    def chunk_body(k, carry):
        c = wid + k * NW
        pltpu.sync_copy(src_hbm.at[pl.ds(c * CH, CH)], sidx)
        pltpu.sync_copy(dst_hbm.at[pl.ds(c * CH, CH)], didx)
        pltpu.async_copy(tab_hbm.at[sidx], rows, gsem).wait()
        for g in range(CH // 16):
            sv = sidx[pl.ds(g * 16, 16)]
            dv = didx[pl.ds(g * 16, 16)]
            ps = plsc.load_gather(pkv, [sv])
            pd = plsc.load_gather(pkv, [dv])
            a = _f16_bits_to_f32(lax.shift_right_logical(ps, 16))
            b = _f16_bits_to_f32(pd)
            e = 1.0 / (1.0 + jnp.exp(-(a + b)))
            ridx = lax.broadcasted_iota(jnp.int32, (16,), 0) + (g * 16)
            for f in range(D):
                fvec = jnp.full((16,), f, jnp.int32)
                col = plsc.load_gather(rows, [ridx, fvec])
                plsc.store_scatter(msg, [ridx, fvec], col * e)
        pltpu.sync_copy(msg, acc.at[didx], add=True)
        return carry

    lax.fori_loop(0, nch, chunk_body, 0)
    plsc.subcore_barrier()
    pltpu.sync_copy(acc.at[pl.ds(row0, rows_per)],
                    out_hbm.at[pl.ds(row0, rows_per)])
    plsc.subcore_barrier()


def _sc_mesh():
    return plsc.VectorSubcoreMesh(core_axis_name="c", subcore_axis_name="s",
                                  num_cores=NC, num_subcores=NS)


def _edge_scratch(NP):
    return [
        pltpu.VMEM_SHARED((NP, GH), jnp.float32),
        pltpu.VMEM((NP,), jnp.int32),
        pltpu.VMEM((CH,), jnp.int32),
        pltpu.VMEM((CH,), jnp.int32),
        pltpu.VMEM((CH, GH), jnp.float32),
        pltpu.VMEM((CH, GH), jnp.float32),
        pltpu.SemaphoreType.DMA,
    ]


_SC_PARAMS = pltpu.CompilerParams(needs_layout_passes=False,
                                  use_tc_tiling_on_sc=False)


def _run_gat1(h1t, h2t, h3t, pks, zeros1, src1d, dst1d, NP, E):
    NCH = E // CH
    rows_per = NP // NS

    def body(h1_hbm, h2_hbm, h3_hbm, pk0, pk1, pk2, zeros_hbm, src_hbm, dst_hbm,
             o0a, o0b, o1a, o1b, o2a, o2b,
             acc, pkv, sidx, didx, rows, msg, gsem):
        cid = lax.axis_index("c")
        sid = lax.axis_index("s")
        wid = sid * NC + cid
        nch = (NCH - wid + NW - 1) // NW
        tabs = (h1_hbm, h2_hbm, h3_hbm)
        pkt = (pk0, pk1, pk2)
        outs = ((o0a, o0b), (o1a, o1b), (o2a, o2b))
        for l in range(3):
            for c in range(NC):
                @pl.when(cid == c)
                def _():
                    _edge_pass(tabs[l], pkt[l], zeros_hbm,
                               src_hbm, dst_hbm, outs[l][c], acc, pkv,
                               sidx, didx, rows, msg, gsem,
                               cid, sid, rows_per, nch, True)

    sds = jax.ShapeDtypeStruct((NP, GH), jnp.float32)
    f = pl.kernel(
        body,
        out_type=[sds] * 6,
        mesh=_sc_mesh(),
        compiler_params=_SC_PARAMS,
        scratch_types=_edge_scratch(NP),
    )
    return f(h1t, h2t, h3t, *pks, zeros1, src1d, dst1d)


def _mid_body(o0a, o0b, o1a, o1b, o2a, o2b, fc2T, a2s, a2d,
              g0, g1, g2, g3, pk):
    h1 = o0a[...] + o0b[...]
    h2 = o1a[...] + o1b[...]
    h3 = o2a[...] + o2b[...]
    h = jnp.concatenate([h1, h2, h3, h3], axis=1)
    h = jnp.where(h > 0, h, jnp.exp(h) - 1.0)
    g = jnp.dot(h, fc2T[...], preferred_element_type=jnp.float32)
    g0[...] = g[:, 0:16]
    g1[...] = g[:, 16:32]
    g2[...] = g[:, 32:48]
    g3[...] = g[:, 48:64]
    a = jnp.dot(g, a2s[...], preferred_element_type=jnp.float32)
    d = jnp.dot(g, a2d[...], preferred_element_type=jnp.float32)
    pk[...] = _pack_logits(a, d)


def _run_mid(g1outs, params, NP):
    grid = (NP // BN,)
    row = lambda w: pl.BlockSpec((BN, w), lambda i: (i, 0))
    full = lambda shape: pl.BlockSpec(shape, lambda i: (0,) * len(shape))
    in_specs = [row(GH)] * 6 + [full((4 * GH, GOUT)), full((GOUT, 1)), full((GOUT, 1))]
    args = list(g1outs) + [
        params['fc2'].T,
        params['att2'][0, :GOUT].reshape(GOUT, 1),
        params['att2'][0, GOUT:].reshape(GOUT, 1),
    ]
    out_shape = (
        [jax.ShapeDtypeStruct((NP, GH), jnp.float32)] * 4
        + [jax.ShapeDtypeStruct((NP, 1), jnp.int32)]
    )
    out_specs = [row(GH)] * 4 + [row(1)]
    return pl.pallas_call(
        _mid_body, grid=grid, in_specs=in_specs, out_specs=out_specs,
        out_shape=out_shape,
    )(*args)


def _run_gat2(gtabs, pk2, zeros1, src1d, dst1d, NP, E):
    NCH = E // CH
    rows_per = NP // NS

    def body(g0, g1, g2, g3, pk_hbm, zeros_hbm, src_hbm, dst_hbm,
             o0a, o0b, o1a, o1b, o2a, o2b, o3a, o3b,
             acc, pkv, sidx, didx, rows, msg, gsem):
        cid = lax.axis_index("c")
        sid = lax.axis_index("s")
        wid = sid * NC + cid
        nch = (NCH - wid + NW - 1) // NW
        tabs = (g0, g1, g2, g3)
        outs = ((o0a, o0b), (o1a, o1b), (o2a, o2b), (o3a, o3b))
        for p in range(4):
            for c in range(NC):
                @pl.when(cid == c)
                def _():
                    _edge_pass(tabs[p], pk_hbm, zeros_hbm,
                               src_hbm, dst_hbm, outs[p][c], acc, pkv,
                               sidx, didx, rows, msg, gsem,
                               cid, sid, rows_per, nch, p == 0)

    sds = jax.ShapeDtypeStruct((NP, GH), jnp.float32)
    f = pl.kernel(
        body,
        out_type=[sds] * 8,
        mesh=_sc_mesh(),
        compiler_params=_SC_PARAMS,
        scratch_types=_edge_scratch(NP),
    )
    return f(*gtabs, pk2, zeros1, src1d, dst1d)


def _run_qgather(g2outs, traj2, q_from, q_to, NP, Q):
    qn = Q // NW

    def body(*refs):
        tabs = refs[0:8]
        t2 = refs[8]
        qf = refs[9]
        qt = refs[10]
        uf = refs[11:19]
        ut = refs[19:27]
        utf = refs[27]
        utt = refs[28]
        qfi = refs[29]
        qti = refs[30]
        buf = refs[31]
        gsem = refs[32]
        cid = lax.axis_index("c")
        sid = lax.axis_index("s")
        wid = sid * NC + cid
        base = wid * qn
        pltpu.sync_copy(qf.at[pl.ds(base, qn)], qfi)
        pltpu.sync_copy(qt.at[pl.ds(base, qn)], qti)
        for i in range(8):
            pltpu.async_copy(tabs[i].at[qfi], buf, gsem).wait()
            pltpu.sync_copy(buf, uf[i].at[pl.ds(base, qn)])
        for i in range(8):
            pltpu.async_copy(tabs[i].at[qti], buf, gsem).wait()
            pltpu.sync_copy(buf, ut[i].at[pl.ds(base, qn)])
        pltpu.async_copy(t2.at[qfi], buf, gsem).wait()
        pltpu.sync_copy(buf, utf.at[pl.ds(base, qn)])
        pltpu.async_copy(t2.at[qti], buf, gsem).wait()
        pltpu.sync_copy(buf, utt.at[pl.ds(base, qn)])

    sds = jax.ShapeDtypeStruct((Q, GH), jnp.float32)
    f = pl.kernel(
        body,
        out_type=[sds] * 18,
        mesh=_sc_mesh(),
        compiler_params=_SC_PARAMS,
        scratch_types=[
            pltpu.VMEM((qn,), jnp.int32),
            pltpu.VMEM((qn,), jnp.int32),
            pltpu.VMEM((qn, GH), jnp.float32),
            pltpu.SemaphoreType.DMA,
        ],
    )
    return f(*g2outs, traj2, q_from, q_to)


def _mlp_body(*refs):
    us = refs[0:18]
    ws = refs[18:36]
    b1 = refs[36]
    w2 = refs[37]
    b2 = refs[38]
    out = refs[39]
    z = b1[...]
    for i in range(18):
        z = z + jnp.dot(us[i][...], ws[i][...], preferred_element_type=jnp.float32)
    z = jnp.maximum(z, 0.0)
    o = jnp.dot(z, w2[...], preferred_element_type=jnp.float32) + b2[...]
    out[...] = jax.nn.sigmoid(o)


def _run_mlp(uouts, params, Q):
    PH = params['pW1'].shape[0]
    W1 = params['pW1']
    ws = []
    for p in range(4):
        for _ in range(NC):
            ws.append(W1[:, 16 * p:16 * (p + 1)].T)
    for p in range(4):
        for _ in range(NC):
            ws.append(W1[:, GOUT + 16 * p:GOUT + 16 * (p + 1)].T)
    ws.append(W1[:, 2 * GOUT:2 * GOUT + HID].T)
    ws.append(W1[:, 2 * GOUT + HID:].T)
    args = list(uouts) + ws + [params['pb1'].reshape(1, PH),
                               params['pW2'].T,
                               params['pb2'].reshape(1, 1)]
    QB = 512 if Q % 512 == 0 else Q
    row = lambda w: pl.BlockSpec((QB, w), lambda i: (i, 0))
    full = lambda a: pl.BlockSpec(a.shape, lambda i: (0,) * a.ndim)
    in_specs = [row(GH)] * 18 + [full(a) for a in args[18:]]
    return pl.pallas_call(
        _mlp_body,
        grid=(Q // QB,),
        in_specs=in_specs,
        out_specs=row(1),
        out_shape=jax.ShapeDtypeStruct((Q, 1), jnp.float32),
    )(*args)


def kernel(x, params, edge_index, q_from, q_to):
    N, T, FEAT = x.shape
    E = edge_index.shape[1]
    Q = q_from.shape[0]
    NP = -(-N // BN) * BN

    xp = jnp.pad(x, ((0, NP - N), (0, 0), (0, 0)))
    xtm = jnp.transpose(xp, (1, 0, 2))
    src1d = edge_index[0]
    dst1d = edge_index[1]

    prep = _run_prep(xtm, params, NP, T, FEAT)
    h1t, h2t, h3t = prep[0:3]
    pks = [p.reshape(NP) for p in prep[3:6]]
    traj2 = prep[6]
    zeros1 = jnp.zeros((NP, GH), jnp.float32)
    g1outs = _run_gat1(h1t, h2t, h3t, pks, zeros1, src1d, dst1d, NP, E)
    mid = _run_mid(g1outs, params, NP)
    gtabs = mid[0:4]
    pk2 = mid[4].reshape(NP)
    g2outs = _run_gat2(gtabs, pk2, zeros1, src1d, dst1d, NP, E)
    uouts = _run_qgather(g2outs, traj2, q_from, q_to, NP, Q)
    return _run_mlp(uouts, params, Q)


# speedup vs baseline: 13.2394x; 2.4862x over previous
import functools

import jax
import jax.numpy as jnp
from jax import lax
from jax.experimental import pallas as pl
from jax.experimental.pallas import tpu as pltpu
from jax.experimental.pallas import tpu_sc as plsc

HID = 16
GH = 16
GOUT = 64
CH = 128
NC = 2
NS = 16
NW = NC * NS
BN = 1024


def _f16_encode(x):
    b = lax.bitcast_convert_type(x, jnp.int32)
    sign = lax.shift_right_logical(b, 16) & jnp.int32(0x8000)
    mag = b & jnp.int32(0x7FFFFFFF)
    mag = jnp.clip(mag, jnp.int32(0x38800000), jnp.int32(0x477FE000))
    em = mag - jnp.int32(0x38000000)
    r = (em + jnp.int32(0x0FFF) + ((em >> 13) & jnp.int32(1))) >> 13
    return sign | r


def _pack_logits(a, d):
    return (_f16_encode(a) << 16) | _f16_encode(d)


def _f16_bits_to_f32(bits):
    sign = (bits & 0x8000) << 16
    em = bits & 0x7FFF
    fb = sign | ((em << 13) + 0x38000000)
    return plsc.bitcast(fb, jnp.float32)


def _prep_body(T, xr_ref, wih0, wih1, wih2, whh0, whh1, whh2,
               bih0, bih1, bih2, bhh0, bhh1, bhh2,
               fcT0, fcT1, fcT2, asv0, asv1, asv2, adv0, adv1, adv2,
               h1t, h2t, h3t, p0, p1, p2, traj2):
    B = xr_ref.shape[2]
    h = [jnp.zeros((HID, B), jnp.float32) for _ in range(3)]
    wih = (wih0, wih1, wih2)
    whh = (whh0, whh1, whh2)
    bih = (bih0, bih1, bih2)
    bhh = (bhh0, bhh1, bhh2)

    def gru_step(inp, hprev, l):
        gi = jnp.dot(wih[l][...], inp, preferred_element_type=jnp.float32) + bih[l][...]
        gh = jnp.dot(whh[l][...], hprev, preferred_element_type=jnp.float32) + bhh[l][...]
        r = jax.nn.sigmoid(gi[0:HID, :] + gh[0:HID, :])
        z = jax.nn.sigmoid(gi[HID:2 * HID, :] + gh[HID:2 * HID, :])
        n = jnp.tanh(gi[2 * HID:, :] + r * gh[2 * HID:, :])
        return (1.0 - z) * n + z * hprev

    for t in range(T):
        inp = xr_ref[:, t, :]
        for l in range(3):
            h[l] = gru_step(inp, h[l], l)
            inp = h[l]

    h = [jnp.transpose(hl, (1, 0)) for hl in h]
    tf = jnp.concatenate(h, axis=1)
    fcT = (fcT0, fcT1, fcT2)
    asv = (asv0, asv1, asv2)
    adv = (adv0, adv1, adv2)
    houts = (h1t, h2t, h3t)
    pouts = (p0, p1, p2)
    for l in range(3):
        H = jnp.dot(tf, fcT[l][...], preferred_element_type=jnp.float32)
        houts[l][...] = H
        a = jnp.dot(H, asv[l][...], preferred_element_type=jnp.float32)
        d = jnp.dot(H, adv[l][...], preferred_element_type=jnp.float32)
        pouts[l][...] = _pack_logits(a, d)
    traj2[...] = h[2]


def _run_prep(xr, params, N, T, FEAT):
    BNP = 1024
    grid = (-(-N // BNP),)
    full = lambda shape: pl.BlockSpec(shape, lambda i: (0,) * len(shape))
    row = lambda w: pl.BlockSpec((BNP, w), lambda i: (i, 0))
    in_specs = [pl.BlockSpec((FEAT, T, BNP), lambda i: (0, 0, i))]
    args = [xr]
    for l in range(3):
        args.append(params['gru_Wih'][l])
        in_specs.append(full((3 * HID, FEAT if l == 0 else HID)))
    for l in range(3):
        args.append(params['gru_Whh'][l])
        in_specs.append(full((3 * HID, HID)))
    for l in range(3):
        args.append(params['gru_bih'][l].reshape(3 * HID, 1))
        in_specs.append(full((3 * HID, 1)))
    for l in range(3):
        args.append(params['gru_bhh'][l].reshape(3 * HID, 1))
        in_specs.append(full((3 * HID, 1)))
    for l in range(3):
        args.append(params['fc'][l].T)
        in_specs.append(full((3 * HID, GH)))
    for l in range(3):
        args.append(params['att'][l][0, :GH].reshape(GH, 1))
        in_specs.append(full((GH, 1)))
    for l in range(3):
        args.append(params['att'][l][0, GH:].reshape(GH, 1))
        in_specs.append(full((GH, 1)))
    out_shape = (
        [jax.ShapeDtypeStruct((N, GH), jnp.float32)] * 3
        + [jax.ShapeDtypeStruct((N, 1), jnp.int32)] * 3
        + [jax.ShapeDtypeStruct((N, HID), jnp.float32)]
    )
    out_specs = [row(GH)] * 3 + [row(1)] * 3 + [row(HID)]
    return pl.pallas_call(
        functools.partial(_prep_body, T),
        grid=grid,
        in_specs=in_specs,
        out_specs=out_specs,
        out_shape=out_shape,
    )(*args)


def _edge_pass(tab_hbm, pk_hbm, zeros_hbm, src_hbm, dst_hbm,
               out_hbm, acc, pkv, sidx, didx, rows, msg, gsem,
               cid, sid, rows_per, nch, load_table):
    D = tab_hbm.shape[1]
    if load_table:
        Ntab = pk_hbm.shape[0]
        if Ntab == pkv.shape[0]:
            pltpu.sync_copy(pk_hbm, pkv)
        else:
            pltpu.sync_copy(pk_hbm, pkv.at[pl.ds(0, Ntab)])
    row0 = sid * rows_per
    pltpu.sync_copy(zeros_hbm.at[pl.ds(row0, rows_per)],
                    acc.at[pl.ds(row0, rows_per)])
    plsc.subcore_barrier()

    wid = sid * NC + cid

    def chunk_body(k, carry):
        c = wid + k * NW
        pltpu.sync_copy(src_hbm.at[pl.ds(c * CH, CH)], sidx)
        pltpu.sync_copy(dst_hbm.at[pl.ds(c * CH, CH)], didx)
        pltpu.async_copy(tab_hbm.at[sidx], rows, gsem).wait()
        for g in range(CH // 16):
            sv = sidx[pl.ds(g * 16, 16)]
            dv = didx[pl.ds(g * 16, 16)]
            ps = plsc.load_gather(pkv, [sv])
            pd = plsc.load_gather(pkv, [dv])
            a = _f16_bits_to_f32(lax.shift_right_logical(ps, 16))
            b = _f16_bits_to_f32(pd)
            e = 1.0 / (1.0 + jnp.exp(-(a + b)))
            ridx = lax.broadcasted_iota(jnp.int32, (16,), 0) + (g * 16)
            fvecs = [jnp.full((16,), f, jnp.int32) for f in range(D)]
            cols = [plsc.load_gather(rows, [ridx, fvecs[f]]) for f in range(D)]
            scaled = [c * e for c in cols]
            for f in range(D):
                plsc.store_scatter(msg, [ridx, fvecs[f]], scaled[f])
        pltpu.sync_copy(msg, acc.at[didx], add=True)
        return carry

    lax.fori_loop(0, nch, chunk_body, 0)
    plsc.subcore_barrier()
    pltpu.sync_copy(acc.at[pl.ds(row0, rows_per)],
                    out_hbm.at[pl.ds(row0, rows_per)])
    plsc.subcore_barrier()


def _sc_mesh():
    return plsc.VectorSubcoreMesh(core_axis_name="c", subcore_axis_name="s",
                                  num_cores=NC, num_subcores=NS)


def _edge_scratch(NP):
    return [
        pltpu.VMEM_SHARED((NP, GH), jnp.float32),
        pltpu.VMEM((NP,), jnp.int32),
        pltpu.VMEM((CH,), jnp.int32),
        pltpu.VMEM((CH,), jnp.int32),
        pltpu.VMEM((CH, GH), jnp.float32),
        pltpu.VMEM((CH, GH), jnp.float32),
        pltpu.SemaphoreType.DMA,
    ]


_SC_PARAMS = pltpu.CompilerParams(needs_layout_passes=False,
                                  use_tc_tiling_on_sc=False)


def _run_gat1(h1t, h2t, h3t, pks, zeros1, src1d, dst1d, NP, E):
    NCH = E // CH
    rows_per = NP // NS

    def body(h1_hbm, h2_hbm, h3_hbm, pk0, pk1, pk2, zeros_hbm, src_hbm, dst_hbm,
             o0a, o0b, o1a, o1b, o2a, o2b,
             acc, pkv, sidx, didx, rows, msg, gsem):
        cid = lax.axis_index("c")
        sid = lax.axis_index("s")
        wid = sid * NC + cid
        nch = (NCH - wid + NW - 1) // NW
        tabs = (h1_hbm, h2_hbm, h3_hbm)
        pkt = (pk0, pk1, pk2)
        outs = ((o0a, o0b), (o1a, o1b), (o2a, o2b))
        for l in range(3):
            for c in range(NC):
                @pl.when(cid == c)
                def _():
                    _edge_pass(tabs[l], pkt[l], zeros_hbm,
                               src_hbm, dst_hbm, outs[l][c], acc, pkv,
                               sidx, didx, rows, msg, gsem,
                               cid, sid, rows_per, nch, True)

    sds = jax.ShapeDtypeStruct((NP, GH), jnp.float32)
    f = pl.kernel(
        body,
        out_type=[sds] * 6,
        mesh=_sc_mesh(),
        compiler_params=_SC_PARAMS,
        scratch_types=_edge_scratch(NP),
    )
    return f(h1t, h2t, h3t, *pks, zeros1, src1d, dst1d)


def _mid_body(o0a, o0b, o1a, o1b, o2a, o2b, fc2T, a2s, a2d,
              g0, g1, g2, g3, pk):
    h1 = o0a[...] + o0b[...]
    h2 = o1a[...] + o1b[...]
    h3 = o2a[...] + o2b[...]
    h = jnp.concatenate([h1, h2, h3, h3], axis=1)
    h = jnp.where(h > 0, h, jnp.exp(h) - 1.0)
    g = jnp.dot(h, fc2T[...], preferred_element_type=jnp.float32)
    g0[...] = g[:, 0:16]
    g1[...] = g[:, 16:32]
    g2[...] = g[:, 32:48]
    g3[...] = g[:, 48:64]
    a = jnp.dot(g, a2s[...], preferred_element_type=jnp.float32)
    d = jnp.dot(g, a2d[...], preferred_element_type=jnp.float32)
    pk[...] = _pack_logits(a, d)


def _run_mid(g1outs, params, NP):
    grid = (NP // BN,)
    row = lambda w: pl.BlockSpec((BN, w), lambda i: (i, 0))
    full = lambda shape: pl.BlockSpec(shape, lambda i: (0,) * len(shape))
    in_specs = [row(GH)] * 6 + [full((4 * GH, GOUT)), full((GOUT, 1)), full((GOUT, 1))]
    args = list(g1outs) + [
        params['fc2'].T,
        params['att2'][0, :GOUT].reshape(GOUT, 1),
        params['att2'][0, GOUT:].reshape(GOUT, 1),
    ]
    out_shape = (
        [jax.ShapeDtypeStruct((NP, GH), jnp.float32)] * 4
        + [jax.ShapeDtypeStruct((NP, 1), jnp.int32)]
    )
    out_specs = [row(GH)] * 4 + [row(1)]
    return pl.pallas_call(
        _mid_body, grid=grid, in_specs=in_specs, out_specs=out_specs,
        out_shape=out_shape,
    )(*args)


def _run_gat2(gtabs, pk2, zeros1, src1d, dst1d, NP, E):
    NCH = E // CH
    rows_per = NP // NS

    def body(g0, g1, g2, g3, pk_hbm, zeros_hbm, src_hbm, dst_hbm,
             o0a, o0b, o1a, o1b, o2a, o2b, o3a, o3b,
             acc, pkv, sidx, didx, rows, msg, gsem):
        cid = lax.axis_index("c")
        sid = lax.axis_index("s")
        wid = sid * NC + cid
        nch = (NCH - wid + NW - 1) // NW
        tabs = (g0, g1, g2, g3)
        outs = ((o0a, o0b), (o1a, o1b), (o2a, o2b), (o3a, o3b))
        for p in range(4):
            for c in range(NC):
                @pl.when(cid == c)
                def _():
                    _edge_pass(tabs[p], pk_hbm, zeros_hbm,
                               src_hbm, dst_hbm, outs[p][c], acc, pkv,
                               sidx, didx, rows, msg, gsem,
                               cid, sid, rows_per, nch, p == 0)

    sds = jax.ShapeDtypeStruct((NP, GH), jnp.float32)
    f = pl.kernel(
        body,
        out_type=[sds] * 8,
        mesh=_sc_mesh(),
        compiler_params=_SC_PARAMS,
        scratch_types=_edge_scratch(NP),
    )
    return f(*gtabs, pk2, zeros1, src1d, dst1d)


def _run_qgather(g2outs, traj2, q_from, q_to, NP, Q):
    qn = Q // NW

    def body(*refs):
        tabs = refs[0:8]
        t2 = refs[8]
        qf = refs[9]
        qt = refs[10]
        uf = refs[11:19]
        ut = refs[19:27]
        utf = refs[27]
        utt = refs[28]
        qfi = refs[29]
        qti = refs[30]
        buf = refs[31]
        gsem = refs[32]
        cid = lax.axis_index("c")
        sid = lax.axis_index("s")
        wid = sid * NC + cid
        base = wid * qn
        pltpu.sync_copy(qf.at[pl.ds(base, qn)], qfi)
        pltpu.sync_copy(qt.at[pl.ds(base, qn)], qti)
        for i in range(8):
            pltpu.async_copy(tabs[i].at[qfi], buf, gsem).wait()
            pltpu.sync_copy(buf, uf[i].at[pl.ds(base, qn)])
        for i in range(8):
            pltpu.async_copy(tabs[i].at[qti], buf, gsem).wait()
            pltpu.sync_copy(buf, ut[i].at[pl.ds(base, qn)])
        pltpu.async_copy(t2.at[qfi], buf, gsem).wait()
        pltpu.sync_copy(buf, utf.at[pl.ds(base, qn)])
        pltpu.async_copy(t2.at[qti], buf, gsem).wait()
        pltpu.sync_copy(buf, utt.at[pl.ds(base, qn)])

    sds = jax.ShapeDtypeStruct((Q, GH), jnp.float32)
    f = pl.kernel(
        body,
        out_type=[sds] * 18,
        mesh=_sc_mesh(),
        compiler_params=_SC_PARAMS,
        scratch_types=[
            pltpu.VMEM((qn,), jnp.int32),
            pltpu.VMEM((qn,), jnp.int32),
            pltpu.VMEM((qn, GH), jnp.float32),
            pltpu.SemaphoreType.DMA,
        ],
    )
    return f(*g2outs, traj2, q_from, q_to)


def _mlp_body(*refs):
    us = refs[0:18]
    ws = refs[18:36]
    b1 = refs[36]
    w2 = refs[37]
    b2 = refs[38]
    out = refs[39]
    z = b1[...]
    for i in range(18):
        z = z + jnp.dot(us[i][...], ws[i][...], preferred_element_type=jnp.float32)
    z = jnp.maximum(z, 0.0)
    o = jnp.dot(z, w2[...], preferred_element_type=jnp.float32) + b2[...]
    out[...] = jax.nn.sigmoid(o)


def _run_mlp(uouts, params, Q):
    PH = params['pW1'].shape[0]
    W1 = params['pW1']
    ws = []
    for p in range(4):
        for _ in range(NC):
            ws.append(W1[:, 16 * p:16 * (p + 1)].T)
    for p in range(4):
        for _ in range(NC):
            ws.append(W1[:, GOUT + 16 * p:GOUT + 16 * (p + 1)].T)
    ws.append(W1[:, 2 * GOUT:2 * GOUT + HID].T)
    ws.append(W1[:, 2 * GOUT + HID:].T)
    args = list(uouts) + ws + [params['pb1'].reshape(1, PH),
                               params['pW2'].T,
                               params['pb2'].reshape(1, 1)]
    QB = 512 if Q % 512 == 0 else Q
    row = lambda w: pl.BlockSpec((QB, w), lambda i: (i, 0))
    full = lambda a: pl.BlockSpec(a.shape, lambda i: (0,) * a.ndim)
    in_specs = [row(GH)] * 18 + [full(a) for a in args[18:]]
    return pl.pallas_call(
        _mlp_body,
        grid=(Q // QB,),
        in_specs=in_specs,
        out_specs=row(1),
        out_shape=jax.ShapeDtypeStruct((Q, 1), jnp.float32),
    )(*args)


def kernel(x, params, edge_index, q_from, q_to):
    N, T, FEAT = x.shape
    E = edge_index.shape[1]
    Q = q_from.shape[0]
    NP = -(-N // BN) * BN

    xr = jnp.transpose(x, (2, 1, 0))
    src1d = edge_index[0]
    dst1d = edge_index[1]

    prep = _run_prep(xr, params, N, T, FEAT)
    h1t, h2t, h3t = prep[0:3]
    pks = [p.reshape(N) for p in prep[3:6]]
    traj2 = prep[6]
    zeros1 = jnp.zeros((NP, GH), jnp.float32)
    g1outs = _run_gat1(h1t, h2t, h3t, pks, zeros1, src1d, dst1d, NP, E)
    mid = _run_mid(g1outs, params, NP)
    gtabs = mid[0:4]
    pk2 = mid[4].reshape(NP)
    g2outs = _run_gat2(gtabs, pk2, zeros1, src1d, dst1d, NP, E)
    uouts = _run_qgather(g2outs, traj2, q_from, q_to, NP, Q)
    return _run_mlp(uouts, params, Q)


# speedup vs baseline: 27.7219x; 2.0939x over previous
import functools

import jax
import jax.numpy as jnp
from jax import lax
from jax.experimental import pallas as pl
from jax.experimental.pallas import tpu as pltpu
from jax.experimental.pallas import tpu_sc as plsc

HID = 16
GH = 16
GOUT = 64
CH = 128
NC = 2
NS = 16
NW = NC * NS
BN = 1024


def _f16_encode(x):
    b = lax.bitcast_convert_type(x, jnp.int32)
    sign = lax.shift_right_logical(b, 16) & jnp.int32(0x8000)
    mag = b & jnp.int32(0x7FFFFFFF)
    mag = jnp.clip(mag, jnp.int32(0x38800000), jnp.int32(0x477FE000))
    em = mag - jnp.int32(0x38000000)
    r = (em + jnp.int32(0x0FFF) + ((em >> 13) & jnp.int32(1))) >> 13
    return sign | r


def _pack_logits(a, d):
    return (_f16_encode(a) << 16) | _f16_encode(d)


def _f16_bits_to_f32(bits):
    sign = (bits & 0x8000) << 16
    em = bits & 0x7FFF
    fb = sign | ((em << 13) + 0x38000000)
    return plsc.bitcast(fb, jnp.float32)


def _prep_body(T, xr_ref, wih0, wih1, wih2, whh0, whh1, whh2,
               bih0, bih1, bih2, bhh0, bhh1, bhh2,
               fcT0, fcT1, fcT2, asv0, asv1, asv2, adv0, adv1, adv2,
               h1t, h2t, h3t, p0, p1, p2, traj2):
    B = xr_ref.shape[2]
    h = [jnp.zeros((HID, B), jnp.float32) for _ in range(3)]
    wih = (wih0, wih1, wih2)
    whh = (whh0, whh1, whh2)
    bih = (bih0, bih1, bih2)
    bhh = (bhh0, bhh1, bhh2)

    def gru_step(inp, hprev, l):
        gi = jnp.dot(wih[l][...], inp, preferred_element_type=jnp.float32) + bih[l][...]
        gh = jnp.dot(whh[l][...], hprev, preferred_element_type=jnp.float32) + bhh[l][...]
        r = jax.nn.sigmoid(gi[0:HID, :] + gh[0:HID, :])
        z = jax.nn.sigmoid(gi[HID:2 * HID, :] + gh[HID:2 * HID, :])
        n = jnp.tanh(gi[2 * HID:, :] + r * gh[2 * HID:, :])
        return (1.0 - z) * n + z * hprev

    for t in range(T):
        inp = xr_ref[:, t, :]
        for l in range(3):
            h[l] = gru_step(inp, h[l], l)
            inp = h[l]

    h = [jnp.transpose(hl, (1, 0)) for hl in h]
    tf = jnp.concatenate(h, axis=1)
    fcT = (fcT0, fcT1, fcT2)
    asv = (asv0, asv1, asv2)
    adv = (adv0, adv1, adv2)
    houts = (h1t, h2t, h3t)
    pouts = (p0, p1, p2)
    for l in range(3):
        H = jnp.dot(tf, fcT[l][...], preferred_element_type=jnp.float32)
        houts[l][...] = H
        a = jnp.dot(H, asv[l][...], preferred_element_type=jnp.float32)
        d = jnp.dot(H, adv[l][...], preferred_element_type=jnp.float32)
        pouts[l][...] = _pack_logits(a, d)
    traj2[...] = h[2]


def _run_prep(xr, params, N, T, FEAT):
    BNP = 1024
    grid = (-(-N // BNP),)
    full = lambda shape: pl.BlockSpec(shape, lambda i: (0,) * len(shape))
    row = lambda w: pl.BlockSpec((BNP, w), lambda i: (i, 0))
    in_specs = [pl.BlockSpec((FEAT, T, BNP), lambda i: (0, 0, i))]
    args = [xr]
    for l in range(3):
        args.append(params['gru_Wih'][l])
        in_specs.append(full((3 * HID, FEAT if l == 0 else HID)))
    for l in range(3):
        args.append(params['gru_Whh'][l])
        in_specs.append(full((3 * HID, HID)))
    for l in range(3):
        args.append(params['gru_bih'][l].reshape(3 * HID, 1))
        in_specs.append(full((3 * HID, 1)))
    for l in range(3):
        args.append(params['gru_bhh'][l].reshape(3 * HID, 1))
        in_specs.append(full((3 * HID, 1)))
    for l in range(3):
        args.append(params['fc'][l].T)
        in_specs.append(full((3 * HID, GH)))
    for l in range(3):
        args.append(params['att'][l][0, :GH].reshape(GH, 1))
        in_specs.append(full((GH, 1)))
    for l in range(3):
        args.append(params['att'][l][0, GH:].reshape(GH, 1))
        in_specs.append(full((GH, 1)))
    out_shape = (
        [jax.ShapeDtypeStruct((N, GH), jnp.float32)] * 3
        + [jax.ShapeDtypeStruct((N, 1), jnp.int32)] * 3
        + [jax.ShapeDtypeStruct((N, HID), jnp.float32)]
    )
    out_specs = [row(GH)] * 3 + [row(1)] * 3 + [row(HID)]
    return pl.pallas_call(
        functools.partial(_prep_body, T),
        grid=grid,
        in_specs=in_specs,
        out_specs=out_specs,
        out_shape=out_shape,
    )(*args)


def _edge_pass(tab_hbm, pk_hbm, zeros_hbm, src_hbm, dst_hbm,
               out_hbm, acc, pkv, sidxb, didxb, rowsb, msgb,
               isem, jsem, gsem, ssem,
               cid, sid, rows_per, nch, load_table):
    D = tab_hbm.shape[1]
    if load_table:
        Ntab = pk_hbm.shape[0]
        if Ntab == pkv.shape[0]:
            pltpu.sync_copy(pk_hbm, pkv)
        else:
            pltpu.sync_copy(pk_hbm, pkv.at[pl.ds(0, Ntab)])
    row0 = sid * rows_per
    pltpu.sync_copy(zeros_hbm.at[pl.ds(row0, rows_per)],
                    acc.at[pl.ds(row0, rows_per)])
    plsc.subcore_barrier()

    wid = sid * NC + cid

    def echunk(k):
        return (wid + k * NW) * CH

    def issue_idx(k, j):
        pltpu.async_copy(src_hbm.at[pl.ds(echunk(k), CH)], sidxb[j], isem[j])
        pltpu.async_copy(dst_hbm.at[pl.ds(echunk(k), CH)], didxb[j], jsem[j])

    def wait_i(j):
        pltpu.make_async_copy(src_hbm.at[pl.ds(0, CH)], sidxb[j], isem[j]).wait()

    def wait_j(j):
        pltpu.make_async_copy(dst_hbm.at[pl.ds(0, CH)], didxb[j], jsem[j]).wait()

    def issue_gather(j):
        pltpu.async_copy(tab_hbm.at[sidxb[j]], rowsb[j], gsem[j])

    def wait_g(j):
        pltpu.make_async_copy(tab_hbm.at[sidxb[j]], rowsb[j], gsem[j]).wait()

    def issue_scatter(j):
        pltpu.async_copy(msgb[j], acc.at[didxb[j]], ssem[j], add=True)

    def wait_s(j):
        pltpu.make_async_copy(msgb[j], acc.at[didxb[j]], ssem[j]).wait()

    def compute(j):
        sidx, didx, rows, msg = sidxb[j], didxb[j], rowsb[j], msgb[j]

        def group(g, carry):
            sv = sidx[pl.ds(g * 16, 16)]
            dv = didx[pl.ds(g * 16, 16)]
            ps = plsc.load_gather(pkv, [sv])
            pd = plsc.load_gather(pkv, [dv])
            a = _f16_bits_to_f32(lax.shift_right_logical(ps, 16))
            b = _f16_bits_to_f32(pd)
            e = 1.0 / (1.0 + jnp.exp(-(a + b)))
            ridx = lax.broadcasted_iota(jnp.int32, (16,), 0) + g * 16
            fvecs = [jnp.full((16,), f, jnp.int32) for f in range(D)]
            cols = [plsc.load_gather(rows, [ridx, fvecs[f]]) for f in range(D)]
            scaled = [c * e for c in cols]
            for f in range(D):
                plsc.store_scatter(msg, [ridx, fvecs[f]], scaled[f])
            return carry

        lax.fori_loop(0, CH // 16, group, 0)

    @pl.when(nch > 0)
    def _():
        issue_idx(0, 0)
        wait_i(0)
        issue_gather(0)

    @pl.when(nch > 1)
    def _():
        issue_idx(1, 1)

    def kk_body(kk, carry):
        for j3 in range(3):
            k = kk * 3 + j3

            @pl.when(k < nch)
            def _():
                r = j3
                rn = (j3 + 1) % 3
                rp = (j3 + 2) % 3

                @pl.when(k + 1 < nch)
                def _():
                    wait_i(rn)
                    issue_gather(rn)

                @pl.when(k + 2 < nch)
                def _():
                    @pl.when(k > 0)
                    def _():
                        wait_s(rp)
                    issue_idx(k + 2, rp)

                wait_g(r)
                wait_j(r)
                compute(r)
                issue_scatter(r)
        return carry

    lax.fori_loop(0, (nch + 2) // 3, kk_body, 0)
    for j in range(3):
        @pl.when(nch > j)
        def _(j=j):
            wait_s(j)
    plsc.subcore_barrier()
    pltpu.sync_copy(acc.at[pl.ds(row0, rows_per)],
                    out_hbm.at[pl.ds(row0, rows_per)])
    plsc.subcore_barrier()


def _sc_mesh():
    return plsc.VectorSubcoreMesh(core_axis_name="c", subcore_axis_name="s",
                                  num_cores=NC, num_subcores=NS)


def _edge_scratch(NP):
    return ([pltpu.VMEM_SHARED((NP, GH), jnp.float32),
             pltpu.VMEM((NP,), jnp.int32)]
            + [pltpu.VMEM((CH,), jnp.int32)] * 6
            + [pltpu.VMEM((CH, GH), jnp.float32)] * 6
            + [pltpu.SemaphoreType.DMA] * 12)


_SC_PARAMS = pltpu.CompilerParams(needs_layout_passes=False,
                                  use_tc_tiling_on_sc=False)


def _run_gat1(h1t, h2t, h3t, pks, zeros1, src1d, dst1d, NP, E):
    NCH = E // CH
    rows_per = NP // NS

    def body(h1_hbm, h2_hbm, h3_hbm, pk0, pk1, pk2, zeros_hbm, src_hbm, dst_hbm,
             o0a, o0b, o1a, o1b, o2a, o2b,
             acc, pkv, si0, si1, si2, di0, di1, di2,
             rw0, rw1, rw2, mg0, mg1, mg2, *sems):
        cid = lax.axis_index("c")
        sid = lax.axis_index("s")
        wid = sid * NC + cid
        nch = (NCH - wid + NW - 1) // NW
        tabs = (h1_hbm, h2_hbm, h3_hbm)
        pkt = (pk0, pk1, pk2)
        outs = ((o0a, o0b), (o1a, o1b), (o2a, o2b))
        for l in range(3):
            for c in range(NC):
                @pl.when(cid == c)
                def _():
                    _edge_pass(tabs[l], pkt[l], zeros_hbm,
                               src_hbm, dst_hbm, outs[l][c], acc, pkv,
                               (si0, si1, si2), (di0, di1, di2),
                               (rw0, rw1, rw2), (mg0, mg1, mg2),
                               sems[0:3], sems[3:6], sems[6:9], sems[9:12],
                               cid, sid, rows_per, nch, True)

    sds = jax.ShapeDtypeStruct((NP, GH), jnp.float32)
    f = pl.kernel(
        body,
        out_type=[sds] * 6,
        mesh=_sc_mesh(),
        compiler_params=_SC_PARAMS,
        scratch_types=_edge_scratch(NP),
    )
    return f(h1t, h2t, h3t, *pks, zeros1, src1d, dst1d)


def _mid_body(o0a, o0b, o1a, o1b, o2a, o2b, fc2T, a2s, a2d,
              g0, g1, g2, g3, pk):
    h1 = o0a[...] + o0b[...]
    h2 = o1a[...] + o1b[...]
    h3 = o2a[...] + o2b[...]
    h = jnp.concatenate([h1, h2, h3, h3], axis=1)
    h = jnp.where(h > 0, h, jnp.exp(h) - 1.0)
    g = jnp.dot(h, fc2T[...], preferred_element_type=jnp.float32)
    g0[...] = g[:, 0:16]
    g1[...] = g[:, 16:32]
    g2[...] = g[:, 32:48]
    g3[...] = g[:, 48:64]
    a = jnp.dot(g, a2s[...], preferred_element_type=jnp.float32)
    d = jnp.dot(g, a2d[...], preferred_element_type=jnp.float32)
    pk[...] = _pack_logits(a, d)


def _run_mid(g1outs, params, NP):
    grid = (NP // BN,)
    row = lambda w: pl.BlockSpec((BN, w), lambda i: (i, 0))
    full = lambda shape: pl.BlockSpec(shape, lambda i: (0,) * len(shape))
    in_specs = [row(GH)] * 6 + [full((4 * GH, GOUT)), full((GOUT, 1)), full((GOUT, 1))]
    args = list(g1outs) + [
        params['fc2'].T,
        params['att2'][0, :GOUT].reshape(GOUT, 1),
        params['att2'][0, GOUT:].reshape(GOUT, 1),
    ]
    out_shape = (
        [jax.ShapeDtypeStruct((NP, GH), jnp.float32)] * 4
        + [jax.ShapeDtypeStruct((NP, 1), jnp.int32)]
    )
    out_specs = [row(GH)] * 4 + [row(1)]
    return pl.pallas_call(
        _mid_body, grid=grid, in_specs=in_specs, out_specs=out_specs,
        out_shape=out_shape,
    )(*args)


def _run_gat2(gtabs, pk2, zeros1, src1d, dst1d, NP, E):
    NCH = E // CH
    rows_per = NP // NS

    def body(g0, g1, g2, g3, pk_hbm, zeros_hbm, src_hbm, dst_hbm,
             o0a, o0b, o1a, o1b, o2a, o2b, o3a, o3b,
             acc, pkv, si0, si1, si2, di0, di1, di2,
             rw0, rw1, rw2, mg0, mg1, mg2, *sems):
        cid = lax.axis_index("c")
        sid = lax.axis_index("s")
        wid = sid * NC + cid
        nch = (NCH - wid + NW - 1) // NW
        tabs = (g0, g1, g2, g3)
        outs = ((o0a, o0b), (o1a, o1b), (o2a, o2b), (o3a, o3b))
        for p in range(4):
            for c in range(NC):
                @pl.when(cid == c)
                def _():
                    _edge_pass(tabs[p], pk_hbm, zeros_hbm,
                               src_hbm, dst_hbm, outs[p][c], acc, pkv,
                               (si0, si1, si2), (di0, di1, di2),
                               (rw0, rw1, rw2), (mg0, mg1, mg2),
                               sems[0:3], sems[3:6], sems[6:9], sems[9:12],
                               cid, sid, rows_per, nch, p == 0)

    sds = jax.ShapeDtypeStruct((NP, GH), jnp.float32)
    f = pl.kernel(
        body,
        out_type=[sds] * 8,
        mesh=_sc_mesh(),
        compiler_params=_SC_PARAMS,
        scratch_types=_edge_scratch(NP),
    )
    return f(*gtabs, pk2, zeros1, src1d, dst1d)


def _run_qgather(g2outs, traj2, q_from, q_to, NP, Q):
    qn = Q // NW

    def body(*refs):
        tabs = refs[0:8]
        t2 = refs[8]
        qf = refs[9]
        qt = refs[10]
        uf = refs[11:19]
        ut = refs[19:27]
        utf = refs[27]
        utt = refs[28]
        qfi = refs[29]
        qti = refs[30]
        buf = refs[31]
        gsem = refs[32]
        cid = lax.axis_index("c")
        sid = lax.axis_index("s")
        wid = sid * NC + cid
        base = wid * qn
        pltpu.sync_copy(qf.at[pl.ds(base, qn)], qfi)
        pltpu.sync_copy(qt.at[pl.ds(base, qn)], qti)
        for i in range(8):
            pltpu.async_copy(tabs[i].at[qfi], buf, gsem).wait()
            pltpu.sync_copy(buf, uf[i].at[pl.ds(base, qn)])
        for i in range(8):
            pltpu.async_copy(tabs[i].at[qti], buf, gsem).wait()
            pltpu.sync_copy(buf, ut[i].at[pl.ds(base, qn)])
        pltpu.async_copy(t2.at[qfi], buf, gsem).wait()
        pltpu.sync_copy(buf, utf.at[pl.ds(base, qn)])
        pltpu.async_copy(t2.at[qti], buf, gsem).wait()
        pltpu.sync_copy(buf, utt.at[pl.ds(base, qn)])

    sds = jax.ShapeDtypeStruct((Q, GH), jnp.float32)
    f = pl.kernel(
        body,
        out_type=[sds] * 18,
        mesh=_sc_mesh(),
        compiler_params=_SC_PARAMS,
        scratch_types=[
            pltpu.VMEM((qn,), jnp.int32),
            pltpu.VMEM((qn,), jnp.int32),
            pltpu.VMEM((qn, GH), jnp.float32),
            pltpu.SemaphoreType.DMA,
        ],
    )
    return f(*g2outs, traj2, q_from, q_to)


def _mlp_body(*refs):
    us = refs[0:18]
    ws = refs[18:36]
    b1 = refs[36]
    w2 = refs[37]
    b2 = refs[38]
    out = refs[39]
    z = b1[...]
    for i in range(18):
        z = z + jnp.dot(us[i][...], ws[i][...], preferred_element_type=jnp.float32)
    z = jnp.maximum(z, 0.0)
    o = jnp.dot(z, w2[...], preferred_element_type=jnp.float32) + b2[...]
    out[...] = jax.nn.sigmoid(o)


def _run_mlp(uouts, params, Q):
    PH = params['pW1'].shape[0]
    W1 = params['pW1']
    ws = []
    for p in range(4):
        for _ in range(NC):
            ws.append(W1[:, 16 * p:16 * (p + 1)].T)
    for p in range(4):
        for _ in range(NC):
            ws.append(W1[:, GOUT + 16 * p:GOUT + 16 * (p + 1)].T)
    ws.append(W1[:, 2 * GOUT:2 * GOUT + HID].T)
    ws.append(W1[:, 2 * GOUT + HID:].T)
    args = list(uouts) + ws + [params['pb1'].reshape(1, PH),
                               params['pW2'].T,
                               params['pb2'].reshape(1, 1)]
    QB = 512 if Q % 512 == 0 else Q
    row = lambda w: pl.BlockSpec((QB, w), lambda i: (i, 0))
    full = lambda a: pl.BlockSpec(a.shape, lambda i: (0,) * a.ndim)
    in_specs = [row(GH)] * 18 + [full(a) for a in args[18:]]
    return pl.pallas_call(
        _mlp_body,
        grid=(Q // QB,),
        in_specs=in_specs,
        out_specs=row(1),
        out_shape=jax.ShapeDtypeStruct((Q, 1), jnp.float32),
    )(*args)


def kernel(x, params, edge_index, q_from, q_to):
    N, T, FEAT = x.shape
    E = edge_index.shape[1]
    Q = q_from.shape[0]
    NP = -(-N // BN) * BN

    xr = jnp.transpose(x, (2, 1, 0))
    src1d = edge_index[0]
    dst1d = edge_index[1]

    prep = _run_prep(xr, params, N, T, FEAT)
    h1t, h2t, h3t = prep[0:3]
    pks = [p.reshape(N) for p in prep[3:6]]
    traj2 = prep[6]
    zeros1 = jnp.zeros((NP, GH), jnp.float32)
    g1outs = _run_gat1(h1t, h2t, h3t, pks, zeros1, src1d, dst1d, NP, E)
    mid = _run_mid(g1outs, params, NP)
    gtabs = mid[0:4]
    pk2 = mid[4].reshape(NP)
    g2outs = _run_gat2(gtabs, pk2, zeros1, src1d, dst1d, NP, E)
    uouts = _run_qgather(g2outs, traj2, q_from, q_to, NP, Q)
    return _run_mlp(uouts, params, Q)


# speedup vs baseline: 29.5800x; 1.0670x over previous
import functools

import jax
import jax.numpy as jnp
from jax import lax
from jax.experimental import pallas as pl
from jax.experimental.pallas import tpu as pltpu
from jax.experimental.pallas import tpu_sc as plsc

HID = 16
GH = 16
GOUT = 64
CH = 128
NC = 2
NS = 16
NW = NC * NS
BN = 1024


def _f16_encode(x):
    b = lax.bitcast_convert_type(x, jnp.int32)
    sign = lax.shift_right_logical(b, 16) & jnp.int32(0x8000)
    mag = b & jnp.int32(0x7FFFFFFF)
    mag = jnp.clip(mag, jnp.int32(0x38800000), jnp.int32(0x477FE000))
    em = mag - jnp.int32(0x38000000)
    r = (em + jnp.int32(0x0FFF) + ((em >> 13) & jnp.int32(1))) >> 13
    return sign | r


def _pack_logits(a, d):
    return (_f16_encode(a) << 16) | _f16_encode(d)


def _f16_bits_to_f32(bits):
    sign = (bits & 0x8000) << 16
    em = bits & 0x7FFF
    fb = sign | ((em << 13) + 0x38000000)
    return plsc.bitcast(fb, jnp.float32)


def _prep_body(T, xr_ref, wih0, wih1, wih2, whh0, whh1, whh2,
               bih0, bih1, bih2, bhh0, bhh1, bhh2,
               fcT0, fcT1, fcT2, asv0, asv1, asv2, adv0, adv1, adv2,
               h1t, h2t, h3t, p0, p1, p2, traj2):
    B = xr_ref.shape[2]
    h = [jnp.zeros((HID, B), jnp.float32) for _ in range(3)]
    wih = (wih0, wih1, wih2)
    whh = (whh0, whh1, whh2)
    bih = (bih0, bih1, bih2)
    bhh = (bhh0, bhh1, bhh2)

    def gru_step(inp, hprev, l):
        gi = jnp.dot(wih[l][...], inp, preferred_element_type=jnp.float32) + bih[l][...]
        gh = jnp.dot(whh[l][...], hprev, preferred_element_type=jnp.float32) + bhh[l][...]
        r = jax.nn.sigmoid(gi[0:HID, :] + gh[0:HID, :])
        z = jax.nn.sigmoid(gi[HID:2 * HID, :] + gh[HID:2 * HID, :])
        n = jnp.tanh(gi[2 * HID:, :] + r * gh[2 * HID:, :])
        return (1.0 - z) * n + z * hprev

    for t in range(T):
        inp = xr_ref[:, t, :]
        for l in range(3):
            h[l] = gru_step(inp, h[l], l)
            inp = h[l]

    h = [jnp.transpose(hl, (1, 0)) for hl in h]
    tf = jnp.concatenate(h, axis=1)
    fcT = (fcT0, fcT1, fcT2)
    asv = (asv0, asv1, asv2)
    adv = (adv0, adv1, adv2)
    houts = (h1t, h2t, h3t)
    pouts = (p0, p1, p2)
    for l in range(3):
        H = jnp.dot(tf, fcT[l][...], preferred_element_type=jnp.float32)
        houts[l][...] = H
        a = jnp.dot(H, asv[l][...], preferred_element_type=jnp.float32)
        d = jnp.dot(H, adv[l][...], preferred_element_type=jnp.float32)
        pouts[l][...] = _pack_logits(a, d)
    traj2[...] = h[2]


def _run_prep(xr, params, N, T, FEAT):
    BNP = 1792
    grid = (-(-N // BNP),)
    full = lambda shape: pl.BlockSpec(shape, lambda i: (0,) * len(shape))
    row = lambda w: pl.BlockSpec((BNP, w), lambda i: (i, 0))
    in_specs = [pl.BlockSpec((FEAT, T, BNP), lambda i: (0, 0, i))]
    args = [xr]
    for l in range(3):
        args.append(params['gru_Wih'][l])
        in_specs.append(full((3 * HID, FEAT if l == 0 else HID)))
    for l in range(3):
        args.append(params['gru_Whh'][l])
        in_specs.append(full((3 * HID, HID)))
    for l in range(3):
        args.append(params['gru_bih'][l].reshape(3 * HID, 1))
        in_specs.append(full((3 * HID, 1)))
    for l in range(3):
        args.append(params['gru_bhh'][l].reshape(3 * HID, 1))
        in_specs.append(full((3 * HID, 1)))
    for l in range(3):
        args.append(params['fc'][l].T)
        in_specs.append(full((3 * HID, GH)))
    for l in range(3):
        args.append(params['att'][l][0, :GH].reshape(GH, 1))
        in_specs.append(full((GH, 1)))
    for l in range(3):
        args.append(params['att'][l][0, GH:].reshape(GH, 1))
        in_specs.append(full((GH, 1)))
    out_shape = (
        [jax.ShapeDtypeStruct((N, GH), jnp.float32)] * 3
        + [jax.ShapeDtypeStruct((N, 1), jnp.int32)] * 3
        + [jax.ShapeDtypeStruct((N, HID), jnp.float32)]
    )
    out_specs = [row(GH)] * 3 + [row(1)] * 3 + [row(HID)]
    return pl.pallas_call(
        functools.partial(_prep_body, T),
        grid=grid,
        in_specs=in_specs,
        out_specs=out_specs,
        out_shape=out_shape,
    )(*args)


def _edge_pass(tab_hbm, pk_hbm, zeros_hbm, src_hbm, dst_hbm,
               out_hbm, acc, pkv, sidxb, didxb, rowsb, msgb,
               isem, jsem, gsem, ssem,
               cid, sid, rows_per, nch, load_table):
    D = tab_hbm.shape[1]
    if load_table:
        Ntab = pk_hbm.shape[0]
        if Ntab == pkv.shape[0]:
            pltpu.sync_copy(pk_hbm, pkv)
        else:
            pltpu.sync_copy(pk_hbm, pkv.at[pl.ds(0, Ntab)])
    row0 = sid * rows_per
    pltpu.sync_copy(zeros_hbm.at[pl.ds(row0, rows_per)],
                    acc.at[pl.ds(row0, rows_per)])
    plsc.subcore_barrier()

    wid = sid * NC + cid

    def echunk(k):
        return (wid + k * NW) * CH

    def issue_idx(k, j):
        pltpu.async_copy(src_hbm.at[pl.ds(echunk(k), CH)], sidxb[j], isem[j])
        pltpu.async_copy(dst_hbm.at[pl.ds(echunk(k), CH)], didxb[j], jsem[j])

    def wait_i(j):
        pltpu.make_async_copy(src_hbm.at[pl.ds(0, CH)], sidxb[j], isem[j]).wait()

    def wait_j(j):
        pltpu.make_async_copy(dst_hbm.at[pl.ds(0, CH)], didxb[j], jsem[j]).wait()

    def issue_gather(j):
        pltpu.async_copy(tab_hbm.at[sidxb[j]], rowsb[j], gsem[j])

    def wait_g(j):
        pltpu.make_async_copy(tab_hbm.at[sidxb[j]], rowsb[j], gsem[j]).wait()

    def issue_scatter(j):
        pltpu.async_copy(msgb[j], acc.at[didxb[j]], ssem[j], add=True)

    def wait_s(j):
        pltpu.make_async_copy(msgb[j], acc.at[didxb[j]], ssem[j]).wait()

    def compute(j):
        sidx, didx, rows, msg = sidxb[j], didxb[j], rowsb[j], msgb[j]

        def group(g, carry):
            sv = sidx[pl.ds(g * 16, 16)]
            dv = didx[pl.ds(g * 16, 16)]
            ps = plsc.load_gather(pkv, [sv])
            pd = plsc.load_gather(pkv, [dv])
            a = _f16_bits_to_f32(lax.shift_right_logical(ps, 16))
            b = _f16_bits_to_f32(pd)
            e = 1.0 / (1.0 + jnp.exp(-(a + b)))
            ridx = lax.broadcasted_iota(jnp.int32, (16,), 0) + g * 16
            fvecs = [jnp.full((16,), f, jnp.int32) for f in range(D)]
            cols = [plsc.load_gather(rows, [ridx, fvecs[f]]) for f in range(D)]
            scaled = [c * e for c in cols]
            for f in range(D):
                plsc.store_scatter(msg, [ridx, fvecs[f]], scaled[f])
            return carry

        lax.fori_loop(0, CH // 16, group, 0)

    @pl.when(nch > 0)
    def _():
        issue_idx(0, 0)
        wait_i(0)
        issue_gather(0)

    @pl.when(nch > 1)
    def _():
        issue_idx(1, 1)

    def kk_body(kk, carry):
        for j3 in range(3):
            k = kk * 3 + j3

            @pl.when(k < nch)
            def _():
                r = j3
                rn = (j3 + 1) % 3
                rp = (j3 + 2) % 3

                @pl.when(k + 1 < nch)
                def _():
                    wait_i(rn)
                    issue_gather(rn)

                @pl.when(k + 2 < nch)
                def _():
                    @pl.when(k > 0)
                    def _():
                        wait_s(rp)
                    issue_idx(k + 2, rp)

                wait_g(r)
                wait_j(r)
                compute(r)
                issue_scatter(r)
        return carry

    lax.fori_loop(0, (nch + 2) // 3, kk_body, 0)
    for j in range(3):
        @pl.when(nch > j)
        def _(j=j):
            wait_s(j)
    plsc.subcore_barrier()
    pltpu.sync_copy(acc.at[pl.ds(row0, rows_per)],
                    out_hbm.at[pl.ds(row0, rows_per)])
    plsc.subcore_barrier()


def _sc_mesh():
    return plsc.VectorSubcoreMesh(core_axis_name="c", subcore_axis_name="s",
                                  num_cores=NC, num_subcores=NS)


def _edge_scratch(NP):
    return ([pltpu.VMEM_SHARED((NP, GH), jnp.float32),
             pltpu.VMEM((NP,), jnp.int32)]
            + [pltpu.VMEM((CH,), jnp.int32)] * 6
            + [pltpu.VMEM((CH, GH), jnp.float32)] * 6
            + [pltpu.SemaphoreType.DMA] * 12)


_SC_PARAMS = pltpu.CompilerParams(needs_layout_passes=False,
                                  use_tc_tiling_on_sc=False)


def _run_gat1(h1t, h2t, h3t, pks, zeros1, src1d, dst1d, NP, E):
    NCH = E // CH
    rows_per = NP // NS

    def body(h1_hbm, h2_hbm, h3_hbm, pk0, pk1, pk2, zeros_hbm, src_hbm, dst_hbm,
             o0a, o0b, o1a, o1b, o2a, o2b,
             acc, pkv, si0, si1, si2, di0, di1, di2,
             rw0, rw1, rw2, mg0, mg1, mg2, *sems):
        cid = lax.axis_index("c")
        sid = lax.axis_index("s")
        wid = sid * NC + cid
        nch = (NCH - wid + NW - 1) // NW
        tabs = (h1_hbm, h2_hbm, h3_hbm)
        pkt = (pk0, pk1, pk2)
        outs = ((o0a, o0b), (o1a, o1b), (o2a, o2b))
        for l in range(3):
            for c in range(NC):
                @pl.when(cid == c)
                def _():
                    _edge_pass(tabs[l], pkt[l], zeros_hbm,
                               src_hbm, dst_hbm, outs[l][c], acc, pkv,
                               (si0, si1, si2), (di0, di1, di2),
                               (rw0, rw1, rw2), (mg0, mg1, mg2),
                               sems[0:3], sems[3:6], sems[6:9], sems[9:12],
                               cid, sid, rows_per, nch, True)

    sds = jax.ShapeDtypeStruct((NP, GH), jnp.float32)
    f = pl.kernel(
        body,
        out_type=[sds] * 6,
        mesh=_sc_mesh(),
        compiler_params=_SC_PARAMS,
        scratch_types=_edge_scratch(NP),
    )
    return f(h1t, h2t, h3t, *pks, zeros1, src1d, dst1d)


def _mid_body(o0a, o0b, o1a, o1b, o2a, o2b, fc2T, a2s, a2d,
              g0, g1, g2, g3, pk):
    h1 = o0a[...] + o0b[...]
    h2 = o1a[...] + o1b[...]
    h3 = o2a[...] + o2b[...]
    h = jnp.concatenate([h1, h2, h3, h3], axis=1)
    h = jnp.where(h > 0, h, jnp.exp(h) - 1.0)
    g = jnp.dot(h, fc2T[...], preferred_element_type=jnp.float32)
    g0[...] = g[:, 0:16]
    g1[...] = g[:, 16:32]
    g2[...] = g[:, 32:48]
    g3[...] = g[:, 48:64]
    a = jnp.dot(g, a2s[...], preferred_element_type=jnp.float32)
    d = jnp.dot(g, a2d[...], preferred_element_type=jnp.float32)
    pk[...] = _pack_logits(a, d)


def _run_mid(g1outs, params, NP):
    grid = (NP // BN,)
    row = lambda w: pl.BlockSpec((BN, w), lambda i: (i, 0))
    full = lambda shape: pl.BlockSpec(shape, lambda i: (0,) * len(shape))
    in_specs = [row(GH)] * 6 + [full((4 * GH, GOUT)), full((GOUT, 1)), full((GOUT, 1))]
    args = list(g1outs) + [
        params['fc2'].T,
        params['att2'][0, :GOUT].reshape(GOUT, 1),
        params['att2'][0, GOUT:].reshape(GOUT, 1),
    ]
    out_shape = (
        [jax.ShapeDtypeStruct((NP, GH), jnp.float32)] * 4
        + [jax.ShapeDtypeStruct((NP, 1), jnp.int32)]
    )
    out_specs = [row(GH)] * 4 + [row(1)]
    return pl.pallas_call(
        _mid_body, grid=grid, in_specs=in_specs, out_specs=out_specs,
        out_shape=out_shape,
    )(*args)


def _run_gat2(gtabs, pk2, zeros1, src1d, dst1d, NP, E):
    NCH = E // CH
    rows_per = NP // NS

    def body(g0, g1, g2, g3, pk_hbm, zeros_hbm, src_hbm, dst_hbm,
             o0a, o0b, o1a, o1b, o2a, o2b, o3a, o3b,
             acc, pkv, si0, si1, si2, di0, di1, di2,
             rw0, rw1, rw2, mg0, mg1, mg2, *sems):
        cid = lax.axis_index("c")
        sid = lax.axis_index("s")
        wid = sid * NC + cid
        nch = (NCH - wid + NW - 1) // NW
        tabs = (g0, g1, g2, g3)
        outs = ((o0a, o0b), (o1a, o1b), (o2a, o2b), (o3a, o3b))
        for p in range(4):
            for c in range(NC):
                @pl.when(cid == c)
                def _():
                    _edge_pass(tabs[p], pk_hbm, zeros_hbm,
                               src_hbm, dst_hbm, outs[p][c], acc, pkv,
                               (si0, si1, si2), (di0, di1, di2),
                               (rw0, rw1, rw2), (mg0, mg1, mg2),
                               sems[0:3], sems[3:6], sems[6:9], sems[9:12],
                               cid, sid, rows_per, nch, p == 0)

    sds = jax.ShapeDtypeStruct((NP, GH), jnp.float32)
    f = pl.kernel(
        body,
        out_type=[sds] * 8,
        mesh=_sc_mesh(),
        compiler_params=_SC_PARAMS,
        scratch_types=_edge_scratch(NP),
    )
    return f(*gtabs, pk2, zeros1, src1d, dst1d)


def _run_qgather(g2outs, traj2, q_from, q_to, NP, Q):
    qn = Q // NW

    def body(*refs):
        tabs = refs[0:8]
        t2 = refs[8]
        qf = refs[9]
        qt = refs[10]
        uf = refs[11:19]
        ut = refs[19:27]
        utf = refs[27]
        utt = refs[28]
        qfi = refs[29]
        qti = refs[30]
        buf = refs[31]
        gsem = refs[32]
        cid = lax.axis_index("c")
        sid = lax.axis_index("s")
        wid = sid * NC + cid
        base = wid * qn
        pltpu.sync_copy(qf.at[pl.ds(base, qn)], qfi)
        pltpu.sync_copy(qt.at[pl.ds(base, qn)], qti)
        for i in range(8):
            pltpu.async_copy(tabs[i].at[qfi], buf, gsem).wait()
            pltpu.sync_copy(buf, uf[i].at[pl.ds(base, qn)])
        for i in range(8):
            pltpu.async_copy(tabs[i].at[qti], buf, gsem).wait()
            pltpu.sync_copy(buf, ut[i].at[pl.ds(base, qn)])
        pltpu.async_copy(t2.at[qfi], buf, gsem).wait()
        pltpu.sync_copy(buf, utf.at[pl.ds(base, qn)])
        pltpu.async_copy(t2.at[qti], buf, gsem).wait()
        pltpu.sync_copy(buf, utt.at[pl.ds(base, qn)])

    sds = jax.ShapeDtypeStruct((Q, GH), jnp.float32)
    f = pl.kernel(
        body,
        out_type=[sds] * 18,
        mesh=_sc_mesh(),
        compiler_params=_SC_PARAMS,
        scratch_types=[
            pltpu.VMEM((qn,), jnp.int32),
            pltpu.VMEM((qn,), jnp.int32),
            pltpu.VMEM((qn, GH), jnp.float32),
            pltpu.SemaphoreType.DMA,
        ],
    )
    return f(*g2outs, traj2, q_from, q_to)


def _mlp_body(*refs):
    us = refs[0:18]
    ws = refs[18:36]
    b1 = refs[36]
    w2 = refs[37]
    b2 = refs[38]
    out = refs[39]
    z = b1[...]
    for i in range(18):
        z = z + jnp.dot(us[i][...], ws[i][...], preferred_element_type=jnp.float32)
    z = jnp.maximum(z, 0.0)
    o = jnp.dot(z, w2[...], preferred_element_type=jnp.float32) + b2[...]
    out[...] = jax.nn.sigmoid(o)


def _run_mlp(uouts, params, Q):
    PH = params['pW1'].shape[0]
    W1 = params['pW1']
    ws = []
    for p in range(4):
        for _ in range(NC):
            ws.append(W1[:, 16 * p:16 * (p + 1)].T)
    for p in range(4):
        for _ in range(NC):
            ws.append(W1[:, GOUT + 16 * p:GOUT + 16 * (p + 1)].T)
    ws.append(W1[:, 2 * GOUT:2 * GOUT + HID].T)
    ws.append(W1[:, 2 * GOUT + HID:].T)
    args = list(uouts) + ws + [params['pb1'].reshape(1, PH),
                               params['pW2'].T,
                               params['pb2'].reshape(1, 1)]
    QB = 512 if Q % 512 == 0 else Q
    row = lambda w: pl.BlockSpec((QB, w), lambda i: (i, 0))
    full = lambda a: pl.BlockSpec(a.shape, lambda i: (0,) * a.ndim)
    in_specs = [row(GH)] * 18 + [full(a) for a in args[18:]]
    return pl.pallas_call(
        _mlp_body,
        grid=(Q // QB,),
        in_specs=in_specs,
        out_specs=row(1),
        out_shape=jax.ShapeDtypeStruct((Q, 1), jnp.float32),
    )(*args)


def kernel(x, params, edge_index, q_from, q_to):
    N, T, FEAT = x.shape
    E = edge_index.shape[1]
    Q = q_from.shape[0]
    NP = -(-N // BN) * BN

    xr = jnp.transpose(x, (2, 1, 0))
    src1d = edge_index[0]
    dst1d = edge_index[1]

    prep = _run_prep(xr, params, N, T, FEAT)
    h1t, h2t, h3t = prep[0:3]
    pks = [p.reshape(N) for p in prep[3:6]]
    traj2 = prep[6]
    zeros1 = jnp.zeros((NP, GH), jnp.float32)
    g1outs = _run_gat1(h1t, h2t, h3t, pks, zeros1, src1d, dst1d, NP, E)
    mid = _run_mid(g1outs, params, NP)
    gtabs = mid[0:4]
    pk2 = mid[4].reshape(NP)
    g2outs = _run_gat2(gtabs, pk2, zeros1, src1d, dst1d, NP, E)
    uouts = _run_qgather(g2outs, traj2, q_from, q_to, NP, Q)
    return _run_mlp(uouts, params, Q)
